# baseline (speedup 1.0000x reference)
import functools

import jax
import jax.numpy as jnp
from jax import lax
from jax.experimental import pallas as pl
from jax.experimental.pallas import tpu as pltpu
from jax.experimental.pallas import tpu_sc as plsc

EMB_DIM = 16
BATCH = 16384
NUM_CORES = 2
NUM_SUBCORES = 16
NUM_WORKERS = NUM_CORES * NUM_SUBCORES
BPW = BATCH // NUM_WORKERS
CHUNK = 128
NCHUNK = BPW // CHUNK
LANES = 16
NVEC = BPW // LANES


def _body(iu_hbm, ii_hbm, ut_hbm, it_hbm, bias_hbm, out_hbm,
          idx_u, idx_i, rows_u, rows_i, out_v, sem_u, sem_i, sem_b):
    wid = lax.axis_index("s") * NUM_CORES + lax.axis_index("c")
    base = wid * BPW

    pltpu.sync_copy(iu_hbm.at[pl.ds(base, BPW)], idx_u)
    pltpu.sync_copy(ii_hbm.at[pl.ds(base, BPW)], idx_i)

    copies = []
    for c in range(NCHUNK):
        sl = pl.ds(c * CHUNK, CHUNK)
        copies.append(pltpu.async_copy(ut_hbm.at[idx_u.at[sl]], rows_u.at[sl], sem_u))
        copies.append(pltpu.async_copy(it_hbm.at[idx_i.at[sl]], rows_i.at[sl], sem_i))
        copies.append(pltpu.async_copy(bias_hbm.at[idx_i.at[sl]], out_v.at[sl], sem_b))
    for cp in copies:
        cp.wait()

    def chunk_body(j, carry):
        rowids = j * LANES + lax.iota(jnp.int32, LANES)
        acc = out_v[pl.ds(j * LANES, LANES)]
        for d in range(EMB_DIM):
            dvec = jnp.full((LANES,), d, jnp.int32)
            uu = plsc.load_gather(rows_u, [rowids, dvec])
            ii = plsc.load_gather(rows_i, [rowids, dvec])
            acc = acc + uu * ii
        out_v[pl.ds(j * LANES, LANES)] = acc
        return carry

    lax.fori_loop(0, NVEC, chunk_body, 0)

    pltpu.sync_copy(out_v, out_hbm.at[pl.ds(base, BPW)])


@jax.jit
def _run(input_user, input_item, u_table, i_table, d_item_bias):
    mesh = plsc.VectorSubcoreMesh(
        core_axis_name="c", subcore_axis_name="s",
        num_cores=NUM_CORES, num_subcores=NUM_SUBCORES)
    f = pl.kernel(
        _body,
        out_type=jax.ShapeDtypeStruct((BATCH,), jnp.float32),
        mesh=mesh,
        scratch_types=[
            pltpu.VMEM((BPW,), jnp.int32),
            pltpu.VMEM((BPW,), jnp.int32),
            pltpu.VMEM((BPW, EMB_DIM), jnp.float32),
            pltpu.VMEM((BPW, EMB_DIM), jnp.float32),
            pltpu.VMEM((BPW,), jnp.float32),
            pltpu.SemaphoreType.DMA,
            pltpu.SemaphoreType.DMA,
            pltpu.SemaphoreType.DMA,
        ],
        compiler_params=pltpu.CompilerParams(
            needs_layout_passes=False, use_tc_tiling_on_sc=False),
    )
    return f(input_user, input_item, u_table, i_table, d_item_bias)


def kernel(input_user, input_item, u_table, i_table, d_item_bias):
    return _run(input_user.astype(jnp.int32), input_item.astype(jnp.int32),
                u_table, i_table, d_item_bias)
